# baseline (speedup 1.0000x reference)
import functools

import jax
import jax.numpy as jnp
from jax import lax
from jax.experimental import pallas as pl
from jax.experimental.pallas import tpu as pltpu
from jax.experimental.pallas import tpu_sc as plsc

E_DIM = 64
N_NEIGHBORS = 16
B = 16384
N_REL = 32

NC = 2
NS = 16
NW = NC * NS
PER_W = B // NW

CHUNK = 64
N_CHUNKS = PER_W // CHUNK
IDX = CHUNK * N_NEIGHBORS
STREAM = 128
N_STREAMS = IDX // STREAM


def _prep_body(ut_ref, et_ref, rt_ref, q_ref, us_ref, es_ref):
    rel = rt_ref[...]
    rn = jnp.sqrt(jnp.sum(rel * rel, axis=-1, keepdims=True))
    rel_n = rel * jnp.minimum(1.0, 1.0 / jnp.maximum(rn, 1e-7))

    u = ut_ref[...]
    un = jnp.sqrt(jnp.sum(u * u, axis=-1, keepdims=True))
    usc = jnp.minimum(1.0, 1.0 / jnp.maximum(un, 1e-7))
    us_ref[...] = usc
    q_ref[...] = lax.dot_general(u * usc, rel_n, (((1,), (1,)), ((), ())))

    e = et_ref[...]
    en = jnp.sqrt(jnp.sum(e * e, axis=-1, keepdims=True))
    es_ref[...] = jnp.minimum(1.0, 1.0 / jnp.maximum(en, 1e-7))


def _prep(user_table, entity_table, relation_table):
    rows = user_table.shape[0]
    blk = 2000
    grid = rows // blk
    return pl.pallas_call(
        _prep_body,
        grid=(grid,),
        in_specs=[
            pl.BlockSpec((blk, E_DIM), lambda i: (i, 0)),
            pl.BlockSpec((blk, E_DIM), lambda i: (i, 0)),
            pl.BlockSpec((N_REL, E_DIM), lambda i: (0, 0)),
        ],
        out_specs=[
            pl.BlockSpec((blk, N_REL), lambda i: (i, 0)),
            pl.BlockSpec((blk, 1), lambda i: (i, 0)),
            pl.BlockSpec((blk, 1), lambda i: (i, 0)),
        ],
        out_shape=[
            jax.ShapeDtypeStruct((rows, N_REL), jnp.float32),
            jax.ShapeDtypeStruct((rows, 1), jnp.float32),
            jax.ShapeDtypeStruct((rows, 1), jnp.float32),
        ],
    )(user_table, entity_table, relation_table)


def _sc_body(users_hbm, items_hbm, adj_e_hbm, adj_r_hbm, user_tab_hbm,
             ent_tab_hbm, q_hbm, uscale_hbm, escale_hbm,
             out_u, out_us, out_sum,
             idx_u, idx_i, pos_adj, nid_flat, rel_flat, pos_q, s_flat,
             wcomb, esc_nbr, rows_v, u_rows, us_v, e0_rows, es0_v, agg_v,
             sem_a, sem_b, sem_c, sem_d):
    wid = lax.axis_index("s") * NC + lax.axis_index("c")
    iota16 = lax.broadcasted_iota(jnp.int32, (16,), 0)

    def chunk_body(t, carry):
        base = wid * PER_W + t * CHUNK
        pltpu.sync_copy(users_hbm.at[pl.ds(base, CHUNK)], idx_u)
        pltpu.sync_copy(items_hbm.at[pl.ds(base, CHUNK)], idx_i)

        h_a = [
            pltpu.async_copy(user_tab_hbm.at[idx_u], u_rows, sem_a),
            pltpu.async_copy(uscale_hbm.at[idx_u], us_v, sem_a),
            pltpu.async_copy(ent_tab_hbm.at[idx_i], e0_rows, sem_a),
            pltpu.async_copy(escale_hbm.at[idx_i], es0_v, sem_a),
        ]

        def padj_body(g, _):
            gofs = pl.multiple_of(g * 16, 16)
            itv16 = idx_i[pl.ds(gofs, 16)] * 16
            for k in range(N_NEIGHBORS):
                pos_adj[pl.ds(gofs + k * CHUNK, 16)] = itv16 + k
            return 0

        lax.fori_loop(0, CHUNK // 16, padj_body, 0)

        h_b = []
        for j in range(N_STREAMS):
            sl = pl.ds(j * STREAM, STREAM)
            h_b.append(pltpu.async_copy(
                adj_e_hbm.at[pos_adj.at[sl]], nid_flat.at[sl], sem_b))
            h_b.append(pltpu.async_copy(
                adj_r_hbm.at[pos_adj.at[sl]], rel_flat.at[sl], sem_b))
        for h in h_b:
            h.wait()

        h_c = []
        for j in range(N_STREAMS):
            sl = pl.ds(j * STREAM, STREAM)
            h_c.append(pltpu.async_copy(
                ent_tab_hbm.at[nid_flat.at[sl]], rows_v.at[sl], sem_c))
            h_c.append(pltpu.async_copy(
                escale_hbm.at[nid_flat.at[sl]], esc_nbr.at[sl], sem_c))

        def pq_body(g, _):
            gofs = pl.multiple_of(g * 16, 16)
            uv32 = idx_u[pl.ds(gofs, 16)] * N_REL
            for k in range(N_NEIGHBORS):
                sl = pl.ds(gofs + k * CHUNK, 16)
                pos_q[sl] = uv32 + rel_flat[sl]
            return 0

        lax.fori_loop(0, CHUNK // 16, pq_body, 0)

        h_d = []
        for j in range(N_STREAMS):
            sl = pl.ds(j * STREAM, STREAM)
            h_d.append(pltpu.async_copy(
                q_hbm.at[pos_q.at[sl]], s_flat.at[sl], sem_d))

        for h in h_a:
            h.wait()
        pltpu.sync_copy(u_rows, out_u.at[pl.ds(base, CHUNK)])
        pltpu.sync_copy(us_v, out_us.at[pl.ds(base, CHUNK)])

        for h in h_c:
            h.wait()
        for h in h_d:
            h.wait()

        def smax_body(g, _):
            gofs = pl.multiple_of(g * 16, 16)
            es = [jnp.exp(s_flat[pl.ds(gofs + k * CHUNK, 16)])
                  for k in range(N_NEIGHBORS)]
            zt = list(es)
            while len(zt) > 1:
                zt = [zt[i] + zt[i + 1] for i in range(0, len(zt), 2)]
            rz = 1.0 / zt[0]
            for k in range(N_NEIGHBORS):
                sl = pl.ds(gofs + k * CHUNK, 16)
                wcomb[sl] = es[k] * (esc_nbr[sl] * rz)
            return 0

        lax.fori_loop(0, CHUNK // 16, smax_body, 0)

        def red_body(g, _):
            gofs = pl.multiple_of(g * 16, 16)
            wk = [wcomb[pl.ds(gofs + k * CHUNK, 16)]
                  for k in range(N_NEIGHBORS)]
            esv = es0_v[pl.ds(gofs, 16)]
            for k16 in range(16):
                c = gofs + k16
                for gg in range(E_DIM // 16):
                    dsl = pl.ds(gg * 16, 16)
                    terms = [wk[k][k16] * rows_v[k * CHUNK + c, dsl]
                             for k in range(N_NEIGHBORS)]
                    terms.append(esv[k16] * e0_rows[c, dsl])
                    while len(terms) > 1:
                        nxt = [terms[i] + terms[i + 1]
                               for i in range(0, len(terms) - 1, 2)]
                        if len(terms) % 2:
                            nxt.append(terms[-1])
                        terms = nxt
                    agg_v[c, dsl] = terms[0]
            return 0

        lax.fori_loop(0, CHUNK // 16, red_body, 0)

        pltpu.sync_copy(agg_v, out_sum.at[pl.ds(base, CHUNK)])
        return carry

    lax.fori_loop(0, N_CHUNKS, chunk_body, 0)


def _sc_all(users_i, items_i, adj_e_flat, adj_r_flat, user_table,
            entity_table, q_flat, uscale, escale):
    mesh = plsc.VectorSubcoreMesh(core_axis_name="c", subcore_axis_name="s")
    f = functools.partial(
        pl.kernel,
        out_type=(
            jax.ShapeDtypeStruct((B, E_DIM), jnp.float32),
            jax.ShapeDtypeStruct((B,), jnp.float32),
            jax.ShapeDtypeStruct((B, E_DIM), jnp.float32),
        ),
        mesh=mesh,
        compiler_params=pltpu.CompilerParams(use_tc_tiling_on_sc=False),
        scratch_types=(
            pltpu.VMEM((CHUNK,), jnp.int32),
            pltpu.VMEM((CHUNK,), jnp.int32),
            pltpu.VMEM((IDX,), jnp.int32),
            pltpu.VMEM((IDX,), jnp.int32),
            pltpu.VMEM((IDX,), jnp.int32),
            pltpu.VMEM((IDX,), jnp.int32),
            pltpu.VMEM((IDX,), jnp.float32),
            pltpu.VMEM((IDX,), jnp.float32),
            pltpu.VMEM((IDX,), jnp.float32),
            pltpu.VMEM((IDX, E_DIM), jnp.float32),
            pltpu.VMEM((CHUNK, E_DIM), jnp.float32),
            pltpu.VMEM((CHUNK,), jnp.float32),
            pltpu.VMEM((CHUNK, E_DIM), jnp.float32),
            pltpu.VMEM((CHUNK,), jnp.float32),
            pltpu.VMEM((CHUNK, E_DIM), jnp.float32),
            pltpu.SemaphoreType.DMA,
            pltpu.SemaphoreType.DMA,
            pltpu.SemaphoreType.DMA,
            pltpu.SemaphoreType.DMA,
        ),
    )(_sc_body)
    return f(users_i, items_i, adj_e_flat, adj_r_flat, user_table,
             entity_table, q_flat, uscale, escale)


def _post_body(u_ref, us_ref, sum_ref, w_ref, b_ref, uout_ref, iout_ref):
    uout_ref[...] = u_ref[...] * us_ref[...]
    out = sum_ref[...] @ w_ref[...] + b_ref[...]
    iout_ref[...] = jnp.tanh(out)


def _post(u_raw, us2, summ, W_agg, b2):
    bt = 1024
    grid = B // bt
    return pl.pallas_call(
        _post_body,
        grid=(grid,),
        in_specs=[
            pl.BlockSpec((bt, E_DIM), lambda i: (i, 0)),
            pl.BlockSpec((bt, 1), lambda i: (i, 0)),
            pl.BlockSpec((bt, E_DIM), lambda i: (i, 0)),
            pl.BlockSpec((E_DIM, E_DIM), lambda i: (0, 0)),
            pl.BlockSpec((1, E_DIM), lambda i: (0, 0)),
        ],
        out_specs=[
            pl.BlockSpec((bt, E_DIM), lambda i: (i, 0)),
            pl.BlockSpec((bt, E_DIM), lambda i: (i, 0)),
        ],
        out_shape=[
            jax.ShapeDtypeStruct((B, E_DIM), jnp.float32),
            jax.ShapeDtypeStruct((B, E_DIM), jnp.float32),
        ],
    )(u_raw, us2, summ, W_agg, b2)


def kernel(users, items, adj_entity, adj_relation, user_table, entity_table,
           relation_table, W_agg, b_agg):
    users_i = users.astype(jnp.int32)
    items_i = items.astype(jnp.int32)
    adj_e_flat = adj_entity.astype(jnp.int32).reshape(-1)
    adj_r_flat = adj_relation.astype(jnp.int32).reshape(-1)

    q_all, uscale, escale = _prep(user_table, entity_table, relation_table)
    u_raw, us_g, summ = _sc_all(
        users_i, items_i, adj_e_flat, adj_r_flat, user_table, entity_table,
        q_all.reshape(-1), uscale.reshape(-1), escale.reshape(-1))
    u_n, item_out = _post(u_raw, us_g.reshape(B, 1), summ, W_agg,
                          b_agg.reshape(1, E_DIM))
    return u_n.reshape(B, 1, E_DIM), item_out


# speedup vs baseline: 1.3141x; 1.3141x over previous
import functools

import jax
import jax.numpy as jnp
from jax import lax
from jax.experimental import pallas as pl
from jax.experimental.pallas import tpu as pltpu
from jax.experimental.pallas import tpu_sc as plsc

E_DIM = 64
N_NEIGHBORS = 16
B = 16384

NC = 2
NS = 16
NW = NC * NS
PER_W = B // NW

CHUNK1 = 64
N_CHUNKS1 = PER_W // CHUNK1
IDX1 = CHUNK1 * N_NEIGHBORS
CHUNK2 = 32
N_CHUNKS2 = PER_W // CHUNK2
IDX2 = CHUNK2 * N_NEIGHBORS
STREAM = 128
N_STREAMS2 = IDX2 // STREAM


def _norm_body(x_ref, o_ref):
    x = x_ref[...]
    n = jnp.sqrt(jnp.sum(x * x, axis=-1, keepdims=True))
    o_ref[...] = jnp.minimum(1.0, 1.0 / jnp.maximum(n, 1e-7))


def _row_scales(table):
    rows = table.shape[0]
    blk = 2000
    grid = rows // blk
    return pl.pallas_call(
        _norm_body,
        grid=(grid,),
        in_specs=[pl.BlockSpec((blk, E_DIM), lambda i: (i, 0))],
        out_specs=pl.BlockSpec((blk, 1), lambda i: (i, 0)),
        out_shape=jax.ShapeDtypeStruct((rows, 1), jnp.float32),
    )(table)


def _sc1_body(users_hbm, items_hbm, adj_e_hbm, adj_r_hbm, user_tab_hbm,
              ent_tab_hbm, out_u, out_e0, out_nid, out_rel,
              idx_u, idx_i, nbr_ids, nbr_flat, rel_buf, rows_u, rows_e0,
              sem_a, sem_b):
    wid = lax.axis_index("s") * NC + lax.axis_index("c")

    def chunk_body(t, carry):
        base = wid * PER_W + t * CHUNK1
        pltpu.sync_copy(users_hbm.at[pl.ds(base, CHUNK1)], idx_u)
        pltpu.sync_copy(items_hbm.at[pl.ds(base, CHUNK1)], idx_i)

        h_ids = pltpu.async_copy(adj_e_hbm.at[idx_i], nbr_ids, sem_a)
        h_rel = pltpu.async_copy(adj_r_hbm.at[idx_i], rel_buf, sem_b)
        h_u = pltpu.async_copy(user_tab_hbm.at[idx_u], rows_u, sem_b)
        h_e0 = pltpu.async_copy(ent_tab_hbm.at[idx_i], rows_e0, sem_b)

        h_ids.wait()

        def flat_body(c, _):
            v = nbr_ids[c, :]
            nbr_flat[pl.ds(pl.multiple_of(c * 16, 16), 16)] = v
            return 0

        lax.fori_loop(0, CHUNK1, flat_body, 0)

        h_rel.wait()
        h_u.wait()
        h_e0.wait()

        pltpu.sync_copy(rows_u, out_u.at[pl.ds(base, CHUNK1)])
        pltpu.sync_copy(rows_e0, out_e0.at[pl.ds(base, CHUNK1)])
        pltpu.sync_copy(nbr_flat, out_nid.at[pl.ds(base * 16, IDX1)])
        pltpu.sync_copy(rel_buf, out_rel.at[pl.ds(base, CHUNK1)])
        return carry

    lax.fori_loop(0, N_CHUNKS1, chunk_body, 0)


def _sc1_gather(users_i, items_i, adj_e, adj_r, user_table, entity_table):
    mesh = plsc.VectorSubcoreMesh(core_axis_name="c", subcore_axis_name="s")
    f = functools.partial(
        pl.kernel,
        out_type=(
            jax.ShapeDtypeStruct((B, E_DIM), jnp.float32),
            jax.ShapeDtypeStruct((B, E_DIM), jnp.float32),
            jax.ShapeDtypeStruct((B * N_NEIGHBORS,), jnp.int32),
            jax.ShapeDtypeStruct((B, N_NEIGHBORS), jnp.int32),
        ),
        mesh=mesh,
        compiler_params=pltpu.CompilerParams(use_tc_tiling_on_sc=False),
        scratch_types=(
            pltpu.VMEM((CHUNK1,), jnp.int32),
            pltpu.VMEM((CHUNK1,), jnp.int32),
            pltpu.VMEM((CHUNK1, N_NEIGHBORS), jnp.int32),
            pltpu.VMEM((IDX1,), jnp.int32),
            pltpu.VMEM((CHUNK1, N_NEIGHBORS), jnp.int32),
            pltpu.VMEM((CHUNK1, E_DIM), jnp.float32),
            pltpu.VMEM((CHUNK1, E_DIM), jnp.float32),
            pltpu.SemaphoreType.DMA,
            pltpu.SemaphoreType.DMA,
        ),
    )(_sc1_body)
    return f(users_i, items_i, adj_e, adj_r, user_table, entity_table)


def _weights_body(u_ref, e0_ref, rel_ref, reltab_ref, uout_ref, e0out_ref,
                  w_ref):
    rel = reltab_ref[...]
    rn = jnp.sqrt(jnp.sum(rel * rel, axis=-1, keepdims=True))
    rel_n = rel * jnp.minimum(1.0, 1.0 / jnp.maximum(rn, 1e-7))

    u = u_ref[...]
    un = jnp.sqrt(jnp.sum(u * u, axis=-1, keepdims=True))
    u_n = u * jnp.minimum(1.0, 1.0 / jnp.maximum(un, 1e-7))
    uout_ref[...] = u_n

    e0 = e0_ref[...]
    en = jnp.sqrt(jnp.sum(e0 * e0, axis=-1, keepdims=True))
    e0out_ref[...] = e0 * jnp.minimum(1.0, 1.0 / jnp.maximum(en, 1e-7))

    p = lax.dot_general(u_n, rel_n, (((1,), (1,)), ((), ())))
    ids = rel_ref[...]
    s = jnp.take_along_axis(p, ids, axis=1)

    m = jnp.max(s, axis=1, keepdims=True)
    e = jnp.exp(s - m)
    w_ref[...] = e / jnp.sum(e, axis=1, keepdims=True)


def _tc_weights(rows_u, rows_e0, rel2, relation_table):
    bt = 512
    grid = B // bt
    return pl.pallas_call(
        _weights_body,
        grid=(grid,),
        in_specs=[
            pl.BlockSpec((bt, E_DIM), lambda i: (i, 0)),
            pl.BlockSpec((bt, E_DIM), lambda i: (i, 0)),
            pl.BlockSpec((bt, N_NEIGHBORS), lambda i: (i, 0)),
            pl.BlockSpec((32, E_DIM), lambda i: (0, 0)),
        ],
        out_specs=[
            pl.BlockSpec((bt, E_DIM), lambda i: (i, 0)),
            pl.BlockSpec((bt, E_DIM), lambda i: (i, 0)),
            pl.BlockSpec((bt, N_NEIGHBORS), lambda i: (i, 0)),
        ],
        out_shape=[
            jax.ShapeDtypeStruct((B, E_DIM), jnp.float32),
            jax.ShapeDtypeStruct((B, E_DIM), jnp.float32),
            jax.ShapeDtypeStruct((B, N_NEIGHBORS), jnp.float32),
        ],
    )(rows_u, rows_e0, rel2, relation_table)


def _sc2_body(nid_hbm, w_hbm, ent_tab_hbm, scale_hbm, out_agg,
              ids0, ids1, w0, w1, sc0, sc1, rows0, rows1, agg0, agg1,
              sem_i0, sem_i1, sem_r0, sem_r1, sem_s0, sem_s1, sem_w0,
              sem_w1):
    wid = lax.axis_index("s") * NC + lax.axis_index("c")
    ids_b = (ids0, ids1)
    w_b = (w0, w1)
    sc_b = (sc0, sc1)
    rows_b = (rows0, rows1)
    agg_b = (agg0, agg1)
    sem_i = (sem_i0, sem_i1)
    sem_r = (sem_r0, sem_r1)
    sem_s = (sem_s0, sem_s1)
    sem_w = (sem_w0, sem_w1)

    def base_of(t):
        return wid * PER_W + t * CHUNK2

    def fire_ids(t, slot):
        b16 = base_of(t) * 16
        pltpu.async_copy(nid_hbm.at[pl.ds(b16, IDX2)], ids_b[slot],
                         sem_i[slot])
        pltpu.async_copy(w_hbm.at[pl.ds(b16, IDX2)], w_b[slot], sem_i[slot])

    def fire_rows(t, slot):
        b16 = base_of(t) * 16
        pltpu.make_async_copy(nid_hbm.at[pl.ds(b16, IDX2)], ids_b[slot],
                              sem_i[slot]).wait()
        pltpu.make_async_copy(w_hbm.at[pl.ds(b16, IDX2)], w_b[slot],
                              sem_i[slot]).wait()
        for j in range(N_STREAMS2):
            sl = pl.ds(j * STREAM, STREAM)
            pltpu.async_copy(
                ent_tab_hbm.at[ids_b[slot].at[sl]], rows_b[slot].at[sl],
                sem_r[slot])
            pltpu.async_copy(
                scale_hbm.at[ids_b[slot].at[sl]], sc_b[slot].at[sl],
                sem_s[slot])

    def drain_rows(slot):
        for j in range(N_STREAMS2):
            sl = pl.ds(j * STREAM, STREAM)
            pltpu.make_async_copy(
                ent_tab_hbm.at[ids_b[slot].at[sl]], rows_b[slot].at[sl],
                sem_r[slot]).wait()
            pltpu.make_async_copy(
                scale_hbm.at[ids_b[slot].at[sl]], sc_b[slot].at[sl],
                sem_s[slot]).wait()

    def finish_rest(t, slot):
        base = base_of(t)

        @pl.when(t >= 2)
        def _():
            pltpu.make_async_copy(
                agg_b[slot], out_agg.at[pl.ds(base, CHUNK2)],
                sem_w[slot]).wait()

        w_v, sc_v, rows_v, agg_v = (w_b[slot], sc_b[slot], rows_b[slot],
                                    agg_b[slot])

        def elem_body(c, _):
            sl16 = pl.ds(pl.multiple_of(c * 16, 16), 16)
            wv = w_v[sl16] * sc_v[sl16]
            for g in range(E_DIM // 16):
                terms = [wv[k] * rows_v[c * 16 + k, pl.ds(g * 16, 16)]
                         for k in range(N_NEIGHBORS)]
                while len(terms) > 1:
                    terms = [terms[i] + terms[i + 1]
                             for i in range(0, len(terms), 2)]
                agg_v[c, pl.ds(g * 16, 16)] = terms[0]
            return 0

        lax.fori_loop(0, CHUNK2, elem_body, 0)
        pltpu.async_copy(agg_v, out_agg.at[pl.ds(base, CHUNK2)], sem_w[slot])

    fire_ids(0, 0)
    fire_rows(0, 0)
    fire_ids(1, 1)

    def pair_body(i, carry):
        t0 = 2 * i
        t1 = t0 + 1
        fire_rows(t1, 1)
        drain_rows(0)
        finish_rest(t0, 0)

        @pl.when(t0 + 2 < N_CHUNKS2)
        def _():
            fire_ids(t0 + 2, 0)

        drain_rows(1)

        @pl.when(t0 + 3 < N_CHUNKS2)
        def _():
            fire_ids(t0 + 3, 1)

        finish_rest(t1, 1)

        @pl.when(t0 + 2 < N_CHUNKS2)
        def _():
            fire_rows(t0 + 2, 0)

        return carry

    lax.fori_loop(0, N_CHUNKS2 // 2, pair_body, 0)

    for slot, t in ((0, N_CHUNKS2 - 2), (1, N_CHUNKS2 - 1)):
        pltpu.make_async_copy(
            agg_b[slot], out_agg.at[pl.ds(base_of(t), CHUNK2)],
            sem_w[slot]).wait()


def _sc2_reduce(nbr_ids_flat, w_flat, entity_table, scales):
    mesh = plsc.VectorSubcoreMesh(core_axis_name="c", subcore_axis_name="s")
    f = functools.partial(
        pl.kernel,
        out_type=jax.ShapeDtypeStruct((B, E_DIM), jnp.float32),
        mesh=mesh,
        compiler_params=pltpu.CompilerParams(use_tc_tiling_on_sc=False),
        scratch_types=(
            pltpu.VMEM((IDX2,), jnp.int32),
            pltpu.VMEM((IDX2,), jnp.int32),
            pltpu.VMEM((IDX2,), jnp.float32),
            pltpu.VMEM((IDX2,), jnp.float32),
            pltpu.VMEM((IDX2,), jnp.float32),
            pltpu.VMEM((IDX2,), jnp.float32),
            pltpu.VMEM((IDX2, E_DIM), jnp.float32),
            pltpu.VMEM((IDX2, E_DIM), jnp.float32),
            pltpu.VMEM((CHUNK2, E_DIM), jnp.float32),
            pltpu.VMEM((CHUNK2, E_DIM), jnp.float32),
            pltpu.SemaphoreType.DMA,
            pltpu.SemaphoreType.DMA,
            pltpu.SemaphoreType.DMA,
            pltpu.SemaphoreType.DMA,
            pltpu.SemaphoreType.DMA,
            pltpu.SemaphoreType.DMA,
            pltpu.SemaphoreType.DMA,
            pltpu.SemaphoreType.DMA,
        ),
    )(_sc2_body)
    return f(nbr_ids_flat, w_flat, entity_table, scales)


def _final_body(e0_ref, agg_ref, w_ref, b_ref, o_ref):
    out = (e0_ref[...] + agg_ref[...]) @ w_ref[...] + b_ref[...]
    o_ref[...] = jnp.tanh(out)


def _tc_final(e0_n, agg, W_agg, b2):
    bt = 1024
    grid = B // bt
    return pl.pallas_call(
        _final_body,
        grid=(grid,),
        in_specs=[
            pl.BlockSpec((bt, E_DIM), lambda i: (i, 0)),
            pl.BlockSpec((bt, E_DIM), lambda i: (i, 0)),
            pl.BlockSpec((E_DIM, E_DIM), lambda i: (0, 0)),
            pl.BlockSpec((1, E_DIM), lambda i: (0, 0)),
        ],
        out_specs=pl.BlockSpec((bt, E_DIM), lambda i: (i, 0)),
        out_shape=jax.ShapeDtypeStruct((B, E_DIM), jnp.float32),
    )(e0_n, agg, W_agg, b2)


def kernel(users, items, adj_entity, adj_relation, user_table, entity_table,
           relation_table, W_agg, b_agg):
    users_i = users.astype(jnp.int32)
    items_i = items.astype(jnp.int32)
    adj_e = adj_entity.astype(jnp.int32)
    adj_r = adj_relation.astype(jnp.int32)

    scales = _row_scales(entity_table).reshape(-1)
    rows_u, rows_e0, nbr_ids_flat, rel2 = _sc1_gather(
        users_i, items_i, adj_e, adj_r, user_table, entity_table)
    u_n, e0_n, w = _tc_weights(rows_u, rows_e0, rel2, relation_table)
    agg = _sc2_reduce(nbr_ids_flat, w.reshape(-1), entity_table, scales)
    item_out = _tc_final(e0_n, agg, W_agg, b_agg.reshape(1, E_DIM))
    return u_n.reshape(B, 1, E_DIM), item_out


# speedup vs baseline: 1.3290x; 1.0114x over previous
import functools

import jax
import jax.numpy as jnp
from jax import lax
from jax.experimental import pallas as pl
from jax.experimental.pallas import tpu as pltpu
from jax.experimental.pallas import tpu_sc as plsc

E_DIM = 64
N_NEIGHBORS = 16
B = 16384

NC = 2
NS = 16
NW = NC * NS
PER_W = B // NW

CHUNK1 = 64
N_CHUNKS1 = PER_W // CHUNK1
IDX1 = CHUNK1 * N_NEIGHBORS
CHUNK2 = 32
N_CHUNKS2 = PER_W // CHUNK2
IDX2 = CHUNK2 * N_NEIGHBORS
STREAM = 128
N_STREAMS2 = IDX2 // STREAM


def _norm_body(x_ref, o_ref):
    x = x_ref[...]
    n = jnp.sqrt(jnp.sum(x * x, axis=-1, keepdims=True))
    o_ref[...] = jnp.minimum(1.0, 1.0 / jnp.maximum(n, 1e-7))


def _row_scales(table):
    rows = table.shape[0]
    blk = 2000
    grid = rows // blk
    return pl.pallas_call(
        _norm_body,
        grid=(grid,),
        in_specs=[pl.BlockSpec((blk, E_DIM), lambda i: (i, 0))],
        out_specs=pl.BlockSpec((blk, 1), lambda i: (i, 0)),
        out_shape=jax.ShapeDtypeStruct((rows, 1), jnp.float32),
    )(table)


def _sc1_body(users_hbm, items_hbm, adj_e_hbm, adj_r_hbm, user_tab_hbm,
              ent_tab_hbm, out_u, out_e0, out_nid, out_rel,
              idx_u, idx_i, nbr_ids, nbr_flat, rel_buf, rows_u, rows_e0,
              sem_a, sem_b):
    wid = lax.axis_index("s") * NC + lax.axis_index("c")

    def chunk_body(t, carry):
        base = wid * PER_W + t * CHUNK1
        pltpu.sync_copy(users_hbm.at[pl.ds(base, CHUNK1)], idx_u)
        pltpu.sync_copy(items_hbm.at[pl.ds(base, CHUNK1)], idx_i)

        h_ids = pltpu.async_copy(adj_e_hbm.at[idx_i], nbr_ids, sem_a)
        h_rel = pltpu.async_copy(adj_r_hbm.at[idx_i], rel_buf, sem_b)
        h_u = pltpu.async_copy(user_tab_hbm.at[idx_u], rows_u, sem_b)
        h_e0 = pltpu.async_copy(ent_tab_hbm.at[idx_i], rows_e0, sem_b)

        h_ids.wait()

        def flat_body(c, _):
            v = nbr_ids[c, :]
            nbr_flat[pl.ds(pl.multiple_of(c * 16, 16), 16)] = v
            return 0

        lax.fori_loop(0, CHUNK1, flat_body, 0)

        h_rel.wait()
        h_u.wait()
        h_e0.wait()

        pltpu.sync_copy(rows_u, out_u.at[pl.ds(base, CHUNK1)])
        pltpu.sync_copy(rows_e0, out_e0.at[pl.ds(base, CHUNK1)])
        pltpu.sync_copy(nbr_flat, out_nid.at[pl.ds(base * 16, IDX1)])
        pltpu.sync_copy(rel_buf, out_rel.at[pl.ds(base, CHUNK1)])
        return carry

    lax.fori_loop(0, N_CHUNKS1, chunk_body, 0)


def _sc1_gather(users_i, items_i, adj_e, adj_r, user_table, entity_table):
    mesh = plsc.VectorSubcoreMesh(core_axis_name="c", subcore_axis_name="s")
    f = functools.partial(
        pl.kernel,
        out_type=(
            jax.ShapeDtypeStruct((B, E_DIM), jnp.float32),
            jax.ShapeDtypeStruct((B, E_DIM), jnp.float32),
            jax.ShapeDtypeStruct((B * N_NEIGHBORS,), jnp.int32),
            jax.ShapeDtypeStruct((B, N_NEIGHBORS), jnp.int32),
        ),
        mesh=mesh,
        compiler_params=pltpu.CompilerParams(use_tc_tiling_on_sc=False),
        scratch_types=(
            pltpu.VMEM((CHUNK1,), jnp.int32),
            pltpu.VMEM((CHUNK1,), jnp.int32),
            pltpu.VMEM((CHUNK1, N_NEIGHBORS), jnp.int32),
            pltpu.VMEM((IDX1,), jnp.int32),
            pltpu.VMEM((CHUNK1, N_NEIGHBORS), jnp.int32),
            pltpu.VMEM((CHUNK1, E_DIM), jnp.float32),
            pltpu.VMEM((CHUNK1, E_DIM), jnp.float32),
            pltpu.SemaphoreType.DMA,
            pltpu.SemaphoreType.DMA,
        ),
    )(_sc1_body)
    return f(users_i, items_i, adj_e, adj_r, user_table, entity_table)


def _weights_body(u_ref, e0_ref, rel_ref, reltab_ref, uout_ref, e0out_ref,
                  w_ref):
    rel = reltab_ref[...]
    rn = jnp.sqrt(jnp.sum(rel * rel, axis=-1, keepdims=True))
    rel_n = rel * jnp.minimum(1.0, 1.0 / jnp.maximum(rn, 1e-7))

    u = u_ref[...]
    un = jnp.sqrt(jnp.sum(u * u, axis=-1, keepdims=True))
    u_n = u * jnp.minimum(1.0, 1.0 / jnp.maximum(un, 1e-7))
    uout_ref[...] = u_n

    e0 = e0_ref[...]
    en = jnp.sqrt(jnp.sum(e0 * e0, axis=-1, keepdims=True))
    e0out_ref[...] = e0 * jnp.minimum(1.0, 1.0 / jnp.maximum(en, 1e-7))

    p = lax.dot_general(u_n, rel_n, (((1,), (1,)), ((), ())))
    ids = rel_ref[...]
    s = jnp.take_along_axis(p, ids, axis=1)

    m = jnp.max(s, axis=1, keepdims=True)
    e = jnp.exp(s - m)
    w_ref[...] = e / jnp.sum(e, axis=1, keepdims=True)


def _tc_weights(rows_u, rows_e0, rel2, relation_table):
    bt = 512
    grid = B // bt
    return pl.pallas_call(
        _weights_body,
        grid=(grid,),
        in_specs=[
            pl.BlockSpec((bt, E_DIM), lambda i: (i, 0)),
            pl.BlockSpec((bt, E_DIM), lambda i: (i, 0)),
            pl.BlockSpec((bt, N_NEIGHBORS), lambda i: (i, 0)),
            pl.BlockSpec((32, E_DIM), lambda i: (0, 0)),
        ],
        out_specs=[
            pl.BlockSpec((bt, E_DIM), lambda i: (i, 0)),
            pl.BlockSpec((bt, E_DIM), lambda i: (i, 0)),
            pl.BlockSpec((bt, N_NEIGHBORS), lambda i: (i, 0)),
        ],
        out_shape=[
            jax.ShapeDtypeStruct((B, E_DIM), jnp.float32),
            jax.ShapeDtypeStruct((B, E_DIM), jnp.float32),
            jax.ShapeDtypeStruct((B, N_NEIGHBORS), jnp.float32),
        ],
    )(rows_u, rows_e0, rel2, relation_table)


def _sc2_body(nid_hbm, w_hbm, ent_tab_hbm, scale_hbm, out_agg,
              ids0, ids1, w0, w1, sc0, sc1, rows0, rows1, agg0, agg1,
              sem_i0, sem_i1, sem_r0, sem_r1, sem_s0, sem_s1, sem_w0,
              sem_w1):
    wid = lax.axis_index("s") * NC + lax.axis_index("c")
    ids_b = (ids0, ids1)
    w_b = (w0, w1)
    sc_b = (sc0, sc1)
    rows_b = (rows0, rows1)
    agg_b = (agg0, agg1)
    sem_i = (sem_i0, sem_i1)
    sem_r = (sem_r0, sem_r1)
    sem_s = (sem_s0, sem_s1)
    sem_w = (sem_w0, sem_w1)

    def base_of(t):
        return wid * PER_W + t * CHUNK2

    def fire_ids(t, slot):
        b16 = base_of(t) * 16
        pltpu.async_copy(nid_hbm.at[pl.ds(b16, IDX2)], ids_b[slot],
                         sem_i[slot])
        pltpu.async_copy(w_hbm.at[pl.ds(b16, IDX2)], w_b[slot], sem_i[slot])

    def fire_rows(t, slot):
        b16 = base_of(t) * 16
        pltpu.make_async_copy(nid_hbm.at[pl.ds(b16, IDX2)], ids_b[slot],
                              sem_i[slot]).wait()
        pltpu.make_async_copy(w_hbm.at[pl.ds(b16, IDX2)], w_b[slot],
                              sem_i[slot]).wait()
        for j in range(N_STREAMS2):
            sl = pl.ds(j * STREAM, STREAM)
            pltpu.async_copy(
                ent_tab_hbm.at[ids_b[slot].at[sl]], rows_b[slot].at[sl],
                sem_r[slot])
            pltpu.async_copy(
                scale_hbm.at[ids_b[slot].at[sl]], sc_b[slot].at[sl],
                sem_s[slot])

    def drain_rows(slot):
        for j in range(N_STREAMS2):
            sl = pl.ds(j * STREAM, STREAM)
            pltpu.make_async_copy(
                ent_tab_hbm.at[ids_b[slot].at[sl]], rows_b[slot].at[sl],
                sem_r[slot]).wait()
            pltpu.make_async_copy(
                scale_hbm.at[ids_b[slot].at[sl]], sc_b[slot].at[sl],
                sem_s[slot]).wait()

    def finish_rest(t, slot):
        base = base_of(t)

        @pl.when(t >= 2)
        def _():
            pltpu.make_async_copy(
                agg_b[slot], out_agg.at[pl.ds(base, CHUNK2)],
                sem_w[slot]).wait()

        w_v, sc_v, rows_v, agg_v = (w_b[slot], sc_b[slot], rows_b[slot],
                                    agg_b[slot])

        def elem_body(c, _):
            sl16 = pl.ds(pl.multiple_of(c * 16, 16), 16)
            wv = w_v[sl16] * sc_v[sl16]
            for g in range(E_DIM // 16):
                terms = [wv[k] * rows_v[c * 16 + k, pl.ds(g * 16, 16)]
                         for k in range(N_NEIGHBORS)]
                while len(terms) > 1:
                    terms = [terms[i] + terms[i + 1]
                             for i in range(0, len(terms), 2)]
                agg_v[c, pl.ds(g * 16, 16)] = terms[0]
            return 0

        lax.fori_loop(0, CHUNK2, elem_body, 0)
        pltpu.async_copy(agg_v, out_agg.at[pl.ds(base, CHUNK2)], sem_w[slot])

    fire_ids(0, 0)
    fire_rows(0, 0)
    fire_ids(1, 1)

    def pair_body(i, carry):
        t0 = 2 * i
        t1 = t0 + 1
        fire_rows(t1, 1)
        drain_rows(0)
        finish_rest(t0, 0)

        @pl.when(t0 + 2 < N_CHUNKS2)
        def _():
            fire_ids(t0 + 2, 0)

        drain_rows(1)
        finish_rest(t1, 1)

        @pl.when(t0 + 3 < N_CHUNKS2)
        def _():
            fire_ids(t0 + 3, 1)

        @pl.when(t0 + 2 < N_CHUNKS2)
        def _():
            fire_rows(t0 + 2, 0)

        return carry

    lax.fori_loop(0, N_CHUNKS2 // 2, pair_body, 0)

    for slot, t in ((0, N_CHUNKS2 - 2), (1, N_CHUNKS2 - 1)):
        pltpu.make_async_copy(
            agg_b[slot], out_agg.at[pl.ds(base_of(t), CHUNK2)],
            sem_w[slot]).wait()


def _sc2_reduce(nbr_ids_flat, w_flat, entity_table, scales):
    mesh = plsc.VectorSubcoreMesh(core_axis_name="c", subcore_axis_name="s")
    f = functools.partial(
        pl.kernel,
        out_type=jax.ShapeDtypeStruct((B, E_DIM), jnp.float32),
        mesh=mesh,
        compiler_params=pltpu.CompilerParams(use_tc_tiling_on_sc=False),
        scratch_types=(
            pltpu.VMEM((IDX2,), jnp.int32),
            pltpu.VMEM((IDX2,), jnp.int32),
            pltpu.VMEM((IDX2,), jnp.float32),
            pltpu.VMEM((IDX2,), jnp.float32),
            pltpu.VMEM((IDX2,), jnp.float32),
            pltpu.VMEM((IDX2,), jnp.float32),
            pltpu.VMEM((IDX2, E_DIM), jnp.float32),
            pltpu.VMEM((IDX2, E_DIM), jnp.float32),
            pltpu.VMEM((CHUNK2, E_DIM), jnp.float32),
            pltpu.VMEM((CHUNK2, E_DIM), jnp.float32),
            pltpu.SemaphoreType.DMA,
            pltpu.SemaphoreType.DMA,
            pltpu.SemaphoreType.DMA,
            pltpu.SemaphoreType.DMA,
            pltpu.SemaphoreType.DMA,
            pltpu.SemaphoreType.DMA,
            pltpu.SemaphoreType.DMA,
            pltpu.SemaphoreType.DMA,
        ),
    )(_sc2_body)
    return f(nbr_ids_flat, w_flat, entity_table, scales)


def _final_body(e0_ref, agg_ref, w_ref, b_ref, o_ref):
    out = (e0_ref[...] + agg_ref[...]) @ w_ref[...] + b_ref[...]
    o_ref[...] = jnp.tanh(out)


def _tc_final(e0_n, agg, W_agg, b2):
    bt = 1024
    grid = B // bt
    return pl.pallas_call(
        _final_body,
        grid=(grid,),
        in_specs=[
            pl.BlockSpec((bt, E_DIM), lambda i: (i, 0)),
            pl.BlockSpec((bt, E_DIM), lambda i: (i, 0)),
            pl.BlockSpec((E_DIM, E_DIM), lambda i: (0, 0)),
            pl.BlockSpec((1, E_DIM), lambda i: (0, 0)),
        ],
        out_specs=pl.BlockSpec((bt, E_DIM), lambda i: (i, 0)),
        out_shape=jax.ShapeDtypeStruct((B, E_DIM), jnp.float32),
    )(e0_n, agg, W_agg, b2)


def kernel(users, items, adj_entity, adj_relation, user_table, entity_table,
           relation_table, W_agg, b_agg):
    users_i = users.astype(jnp.int32)
    items_i = items.astype(jnp.int32)
    adj_e = adj_entity.astype(jnp.int32)
    adj_r = adj_relation.astype(jnp.int32)

    scales = _row_scales(entity_table).reshape(-1)
    rows_u, rows_e0, nbr_ids_flat, rel2 = _sc1_gather(
        users_i, items_i, adj_e, adj_r, user_table, entity_table)
    u_n, e0_n, w = _tc_weights(rows_u, rows_e0, rel2, relation_table)
    agg = _sc2_reduce(nbr_ids_flat, w.reshape(-1), entity_table, scales)
    item_out = _tc_final(e0_n, agg, W_agg, b_agg.reshape(1, E_DIM))
    return u_n.reshape(B, 1, E_DIM), item_out
